# baseline (speedup 1.0000x reference)
import functools

import jax
import jax.numpy as jnp
from jax import lax
from jax.experimental import pallas as pl
from jax.experimental.pallas import tpu as pltpu
from jax.experimental.pallas import tpu_sc as plsc

_BATCH = 16384
_NF = 26
_D = 64
_NW = 32
_BPW = _BATCH // _NW
_QB = _BPW // 128
_NCHUNK = _QB * _NF
_NBUF = 4


def _emb_body(idxt_hbm, table_hbm, out_hbm, idx_v, rows_v, *sems):
    gsem = sems[:_NBUF]
    osem = sems[_NBUF:]
    wid = lax.axis_index("s") * 2 + lax.axis_index("c")
    b0 = wid * _BPW
    pltpu.sync_copy(idxt_hbm.at[:, pl.ds(b0, _BPW)], idx_v)

    def gather(c, b):
        q = c // _NF
        f = c % _NF
        return pltpu.make_async_copy(
            table_hbm.at[idx_v.at[f, pl.ds(q * 128, 128)]],
            rows_v.at[b],
            gsem[b],
        )

    def outwrite(c, b):
        q = c // _NF
        f = c % _NF
        return pltpu.make_async_copy(
            rows_v.at[b, :, pl.ds(0, _D)],
            out_hbm.at[pl.ds(b0 + q * 128, 128), f],
            osem[b],
        )

    for b in range(_NBUF):
        gather(b, b).start()

    def body(g, carry):
        g0 = g * _NBUF
        for b in range(_NBUF):
            c = g0 + b
            gather(c, b).wait()
            outwrite(c, b).start()
        for b in range(_NBUF):
            cn = g0 + _NBUF + b

            @pl.when(cn < _NCHUNK)
            def _():
                outwrite(cn - _NBUF, b).wait()
                gather(cn, b).start()
        return carry

    lax.fori_loop(0, _NCHUNK // _NBUF, body, 0)

    for b in range(_NBUF):
        outwrite(_NCHUNK - _NBUF + b, b).wait()


@jax.jit
def kernel(indices, table):
    mesh = plsc.VectorSubcoreMesh(core_axis_name="c", subcore_axis_name="s")
    run = functools.partial(
        pl.kernel,
        out_type=jax.ShapeDtypeStruct((_BATCH, _NF, _D), jnp.float32),
        mesh=mesh,
        scratch_types=[
            pltpu.VMEM((_NF, _BPW), jnp.int32),
            pltpu.VMEM((_NBUF, 128, 128), jnp.float32),
        ]
        + [pltpu.SemaphoreType.DMA] * (2 * _NBUF),
        compiler_params=pltpu.CompilerParams(use_tc_tiling_on_sc=False),
    )(_emb_body)
    return run(indices.T, jnp.pad(table, ((0, 0), (0, 64))))


# speedup vs baseline: 1.1222x; 1.1222x over previous
import functools

import jax
import jax.numpy as jnp
from jax import lax
from jax.experimental import pallas as pl
from jax.experimental.pallas import tpu as pltpu
from jax.experimental.pallas import tpu_sc as plsc

_BATCH = 16384
_NF = 26
_D = 64
_V = 1_000_000
_NW = 32
_BPW = _BATCH // _NW
_QB = _BPW // 128
_NCHUNK = _QB * _NF
_NBUF = 4

_VCOLS = _V // 128
_STEPS = _VCOLS // (2 * _NW)
_TAIL = _V - _VCOLS * 128


def _fmt_body(tabt_hbm, tail_hbm, flat_hbm, src_v, dst0, dst1,
              rs0, rs1, ws0, ws1):
    dst = (dst0, dst1)
    rsem = (rs0, rs1)
    wsem = (ws0, ws1)
    wid = lax.axis_index("s") * 2 + lax.axis_index("c")
    iota16 = lax.iota(jnp.int32, 16)
    rot = [(iota16 + kk) & 15 for kk in range(16)]
    rotd = [r * _D + iota16 for r in rot]

    def read_start(k, b):
        v0 = (k * _NW + wid) * 256
        pltpu.make_async_copy(
            tabt_hbm.at[:, pl.ds(v0, 128)],
            src_v.at[b, pl.ds(0, _D)],
            rsem[b],
        ).start()
        pltpu.make_async_copy(
            tabt_hbm.at[:, pl.ds(v0 + 128, 128)],
            src_v.at[b, pl.ds(_D, _D)],
            rsem[b],
        ).start()

    def read_wait(k, b):
        v0 = (k * _NW + wid) * 256
        pltpu.make_async_copy(
            tabt_hbm.at[:, pl.ds(v0, 128)],
            src_v.at[b, pl.ds(0, _D)],
            rsem[b],
        ).wait()
        pltpu.make_async_copy(
            tabt_hbm.at[:, pl.ds(v0 + 128, 128)],
            src_v.at[b, pl.ds(_D, _D)],
            rsem[b],
        ).wait()

    def write(k, b):
        v0 = (k * _NW + wid) * 256
        return pltpu.make_async_copy(
            dst[b], flat_hbm.at[pl.ds(v0 * _D, 256 * _D)], wsem[b]
        )

    def transpose(b):
        def strip(vb, carry):
            h = vb // 8
            v0 = (vb % 8) * 16
            for db in range(4):
                d0 = db * 16
                rows = h * _D + d0 + iota16
                base = vb * (16 * _D) + d0
                for kk in range(16):
                    val = plsc.load_gather(src_v.at[b], [rows, v0 + rot[kk]])
                    plsc.store_scatter(dst[b], [base + rotd[kk]], val)
            return carry

        lax.fori_loop(0, 16, strip, 0)

    read_start(0, 0)

    def body(g, carry):
        for b in range(2):
            k = g * 2 + b
            kn = k + 1

            @pl.when(kn < _STEPS)
            def _():
                read_start(kn, 1 - b)

            read_wait(k, b)

            @pl.when(k >= 2)
            def _():
                write(k - 2, b).wait()

            transpose(b)
            write(k, b).start()
        return carry

    lax.fori_loop(0, _STEPS // 2, body, 0)
    write(_STEPS - 2, 0).wait()
    write(_STEPS - 1, 1).wait()

    @pl.when(wid < 4)
    def _():
        c = _STEPS * _NW * 2 + wid
        pltpu.sync_copy(
            tabt_hbm.at[:, pl.ds(c * 128, 128)], src_v.at[0, pl.ds(0, _D)]
        )

        def strip(vb, carry):
            v0 = vb * 16
            for db in range(4):
                d0 = db * 16
                rows = d0 + iota16
                base = vb * (16 * _D) + d0
                for kk in range(16):
                    val = plsc.load_gather(src_v.at[0], [rows, v0 + rot[kk]])
                    plsc.store_scatter(dst0, [base + rotd[kk]], val)
            return carry

        lax.fori_loop(0, 8, strip, 0)
        pltpu.sync_copy(
            dst0.at[pl.ds(0, 128 * _D)],
            flat_hbm.at[pl.ds(c * 128 * _D, 128 * _D)],
        )

    @pl.when(wid == 4)
    def _():
        pltpu.sync_copy(tail_hbm, dst1.at[pl.ds(0, _TAIL * _D)])
        pltpu.sync_copy(
            dst1.at[pl.ds(0, _TAIL * _D)],
            flat_hbm.at[pl.ds((_V - _TAIL) * _D, _TAIL * _D)],
        )


def _emb_body(idxt_hbm, table_hbm, out_hbm, idx_v, rows_v, *sems):
    gsem = sems[:_NBUF]
    osem = sems[_NBUF:]
    wid = lax.axis_index("s") * 2 + lax.axis_index("c")
    b0 = wid * _BPW
    pltpu.sync_copy(idxt_hbm.at[:, pl.ds(b0, _BPW)], idx_v)

    def gather(c, b):
        q = c // _NF
        f = c % _NF
        return pltpu.make_async_copy(
            table_hbm.at[idx_v.at[f, pl.ds(q * 128, 128)]],
            rows_v.at[b],
            gsem[b],
        )

    def outwrite(c, b):
        q = c // _NF
        f = c % _NF
        return pltpu.make_async_copy(
            rows_v.at[b],
            out_hbm.at[pl.ds(b0 + q * 128, 128), f],
            osem[b],
        )

    for b in range(_NBUF):
        gather(b, b).start()

    def body(g, carry):
        g0 = g * _NBUF
        for b in range(_NBUF):
            c = g0 + b
            gather(c, b).wait()
            outwrite(c, b).start()
        for b in range(_NBUF):
            cn = g0 + _NBUF + b

            @pl.when(cn < _NCHUNK)
            def _():
                outwrite(cn - _NBUF, b).wait()
                gather(cn, b).start()
        return carry

    lax.fori_loop(0, _NCHUNK // _NBUF, body, 0)
    for b in range(_NBUF):
        outwrite(_NCHUNK - _NBUF + b, b).wait()


@jax.jit
def kernel(indices, table):
    mesh = plsc.VectorSubcoreMesh(core_axis_name="c", subcore_axis_name="s")
    fmt = functools.partial(
        pl.kernel,
        out_type=jax.ShapeDtypeStruct((_V * _D,), jnp.float32),
        mesh=mesh,
        scratch_types=[
            pltpu.VMEM((2, 2 * _D, 128), jnp.float32),
            pltpu.VMEM((256 * _D,), jnp.float32),
            pltpu.VMEM((256 * _D,), jnp.float32),
        ]
        + [pltpu.SemaphoreType.DMA] * 4,
        compiler_params=pltpu.CompilerParams(
            use_tc_tiling_on_sc=True, needs_layout_passes=False
        ),
    )(_fmt_body)
    tail = table[_V - _TAIL :, :].reshape(_TAIL * _D)
    flat = fmt(table.T, tail)

    run = functools.partial(
        pl.kernel,
        out_type=jax.ShapeDtypeStruct((_BATCH, _NF, _D), jnp.float32),
        mesh=mesh,
        scratch_types=[
            pltpu.VMEM((_NF, _BPW), jnp.int32),
            pltpu.VMEM((_NBUF, 128, _D), jnp.float32),
        ]
        + [pltpu.SemaphoreType.DMA] * (2 * _NBUF),
        compiler_params=pltpu.CompilerParams(use_tc_tiling_on_sc=False),
    )(_emb_body)
    return run(indices.T, flat.reshape(_V, _D))


# speedup vs baseline: 1.1850x; 1.0559x over previous
import functools

import jax
import jax.numpy as jnp
from jax import lax
from jax.experimental import pallas as pl
from jax.experimental.pallas import tpu as pltpu
from jax.experimental.pallas import tpu_sc as plsc

_BATCH = 16384
_NF = 26
_D = 64
_V = 1_000_000
_NW = 32
_BPW = _BATCH // _NW
_QB = _BPW // 128
_NCHUNK = _QB * _NF
_NBUF = 4

_VCOLS = _V // 128
_STEPS = _VCOLS // (2 * _NW)
_TAIL = _V - _VCOLS * 128


def _fmt_body(tabt_hbm, tail_hbm, flat_hbm, src_v, dst0, dst1,
              rs0, rs1, ws0, ws1):
    dst = (dst0, dst1)
    rsem = (rs0, rs1)
    wsem = (ws0, ws1)
    wid = lax.axis_index("s") * 2 + lax.axis_index("c")
    iota16 = lax.iota(jnp.int32, 16)
    rot = [(iota16 + kk) & 15 for kk in range(16)]
    rotd = [r * _D + iota16 for r in rot]

    def read_start(k, b):
        v0 = (k * _NW + wid) * 256
        pltpu.make_async_copy(
            tabt_hbm.at[:, pl.ds(v0, 128)],
            src_v.at[b, pl.ds(0, _D)],
            rsem[b],
        ).start()
        pltpu.make_async_copy(
            tabt_hbm.at[:, pl.ds(v0 + 128, 128)],
            src_v.at[b, pl.ds(_D, _D)],
            rsem[b],
        ).start()

    def read_wait(k, b):
        v0 = (k * _NW + wid) * 256
        pltpu.make_async_copy(
            tabt_hbm.at[:, pl.ds(v0, 128)],
            src_v.at[b, pl.ds(0, _D)],
            rsem[b],
        ).wait()
        pltpu.make_async_copy(
            tabt_hbm.at[:, pl.ds(v0 + 128, 128)],
            src_v.at[b, pl.ds(_D, _D)],
            rsem[b],
        ).wait()

    def write(k, b):
        v0 = (k * _NW + wid) * 256
        return pltpu.make_async_copy(
            dst[b], flat_hbm.at[pl.ds(v0 * _D, 256 * _D)], wsem[b]
        )

    def transpose(b):
        def quad(q, carry):
            for i in range(4):
                vb = q * 4 + i
                hh = vb // 8
                v0 = (vb % 8) * 16
                for db in range(4):
                    d0 = db * 16
                    rows = hh * _D + d0 + iota16
                    base = vb * (16 * _D) + d0
                    for kk in range(16):
                        val = plsc.load_gather(
                            src_v.at[b], [rows, v0 + rot[kk]]
                        )
                        plsc.store_scatter(dst[b], [base + rotd[kk]], val)
            return carry

        lax.fori_loop(0, 4, quad, 0)

    read_start(0, 0)

    def body(g, carry):
        for b in range(2):
            k = g * 2 + b
            kn = k + 1

            @pl.when(kn < _STEPS)
            def _():
                read_start(kn, 1 - b)

            read_wait(k, b)

            @pl.when(k >= 2)
            def _():
                write(k - 2, b).wait()

            transpose(b)
            write(k, b).start()
        return carry

    lax.fori_loop(0, _STEPS // 2, body, 0)
    write(_STEPS - 2, 0).wait()
    write(_STEPS - 1, 1).wait()

    @pl.when(wid < 4)
    def _():
        c = _STEPS * _NW * 2 + wid
        pltpu.sync_copy(
            tabt_hbm.at[:, pl.ds(c * 128, 128)], src_v.at[0, pl.ds(0, _D)]
        )

        def strip(vb, carry):
            v0 = vb * 16
            for db in range(4):
                d0 = db * 16
                rows = d0 + iota16
                base = vb * (16 * _D) + d0
                for kk in range(16):
                    val = plsc.load_gather(src_v.at[0], [rows, v0 + rot[kk]])
                    plsc.store_scatter(dst0, [base + rotd[kk]], val)
            return carry

        lax.fori_loop(0, 8, strip, 0)
        pltpu.sync_copy(
            dst0.at[pl.ds(0, 128 * _D)],
            flat_hbm.at[pl.ds(c * 128 * _D, 128 * _D)],
        )

    @pl.when(wid == 4)
    def _():
        pltpu.sync_copy(tail_hbm, dst1.at[pl.ds(0, _TAIL * _D)])
        pltpu.sync_copy(
            dst1.at[pl.ds(0, _TAIL * _D)],
            flat_hbm.at[pl.ds((_V - _TAIL) * _D, _TAIL * _D)],
        )


def _emb_body(idxt_hbm, table_hbm, out_hbm, idx_v, rows_v, *sems):
    gsem = sems[:_NBUF]
    osem = sems[_NBUF:]
    wid = lax.axis_index("s") * 2 + lax.axis_index("c")
    b0 = wid * _BPW
    pltpu.sync_copy(idxt_hbm.at[:, pl.ds(b0, _BPW)], idx_v)

    def gather(c, b):
        q = c // _NF
        f = c % _NF
        return pltpu.make_async_copy(
            table_hbm.at[idx_v.at[f, pl.ds(q * 128, 128)]],
            rows_v.at[b],
            gsem[b],
        )

    def outwrite(c, b):
        q = c // _NF
        f = c % _NF
        return pltpu.make_async_copy(
            rows_v.at[b],
            out_hbm.at[pl.ds(b0 + q * 128, 128), f],
            osem[b],
        )

    for b in range(_NBUF):
        gather(b, b).start()

    def body(g, carry):
        g0 = g * _NBUF
        for b in range(_NBUF):
            c = g0 + b
            gather(c, b).wait()
            outwrite(c, b).start()
        for b in range(_NBUF):
            cn = g0 + _NBUF + b

            @pl.when(cn < _NCHUNK)
            def _():
                outwrite(cn - _NBUF, b).wait()
                gather(cn, b).start()
        return carry

    lax.fori_loop(0, _NCHUNK // _NBUF, body, 0)
    for b in range(_NBUF):
        outwrite(_NCHUNK - _NBUF + b, b).wait()


@jax.jit
def kernel(indices, table):
    mesh = plsc.VectorSubcoreMesh(core_axis_name="c", subcore_axis_name="s")
    fmt = functools.partial(
        pl.kernel,
        out_type=jax.ShapeDtypeStruct((_V * _D,), jnp.float32),
        mesh=mesh,
        scratch_types=[
            pltpu.VMEM((2, 2 * _D, 128), jnp.float32),
            pltpu.VMEM((256 * _D,), jnp.float32),
            pltpu.VMEM((256 * _D,), jnp.float32),
        ]
        + [pltpu.SemaphoreType.DMA] * 4,
        compiler_params=pltpu.CompilerParams(
            use_tc_tiling_on_sc=True, needs_layout_passes=False
        ),
    )(_fmt_body)
    tail = table[_V - _TAIL :, :].reshape(_TAIL * _D)
    flat = fmt(table.T, tail)

    run = functools.partial(
        pl.kernel,
        out_type=jax.ShapeDtypeStruct((_BATCH, _NF, _D), jnp.float32),
        mesh=mesh,
        scratch_types=[
            pltpu.VMEM((_NF, _BPW), jnp.int32),
            pltpu.VMEM((_NBUF, 128, _D), jnp.float32),
        ]
        + [pltpu.SemaphoreType.DMA] * (2 * _NBUF),
        compiler_params=pltpu.CompilerParams(use_tc_tiling_on_sc=False),
    )(_emb_body)
    return run(indices.T, flat.reshape(_V, _D))
